# Initial kernel scaffold; baseline (speedup 1.0000x reference)
import jax
import jax.numpy as jnp
from jax.experimental import pallas as pl


def kernel(uids, user_embedding, attr_table, embed_tables, W, b):
    raise NotImplementedError("write your pallas kernel here")


# speedup vs baseline: 2.1211x; 2.1211x over previous
import functools

import jax
import jax.numpy as jnp
from jax import lax
from jax.experimental import pallas as pl
from jax.experimental.pallas import tpu as pltpu
from jax.experimental.pallas import tpu_sc as plsc

B = 4096
C = 26
V = 100000
H = 32
NU = 100000

NC = 2
NS = 16
L = 16
NW = NC * NS
BPW = B // NW

_mesh = plsc.VectorSubcoreMesh(core_axis_name="c", subcore_axis_name="s")


@functools.partial(
    pl.kernel,
    mesh=_mesh,
    out_type=jax.ShapeDtypeStruct((C, B, H), jnp.float32),
    scratch_types=[
        pltpu.VMEM((BPW,), jnp.int32),
        pltpu.VMEM((C, BPW), jnp.int32),
        pltpu.VMEM((C, BPW), jnp.int32),
        pltpu.VMEM((C, BPW, H), jnp.float32),
        pltpu.SemaphoreType.DMA,
    ],
    compiler_params=pltpu.CompilerParams(use_tc_tiling_on_sc=False),
)
def _sc_gather(uids_hbm, attr_flat_hbm, emb2d_hbm, out_hbm,
               uids_v, idx1, gidx, rows, sem):
    wid = lax.axis_index("s") * NC + lax.axis_index("c")
    base = wid * BPW
    pltpu.sync_copy(uids_hbm.at[pl.ds(base, BPW)], uids_v)

    for i in range(BPW // L):
        u = uids_v[pl.ds(i * L, L)] * C
        for c in range(C):
            idx1[c, pl.ds(i * L, L)] = u + c

    cps = [pltpu.async_copy(attr_flat_hbm.at[idx1.at[c]], gidx.at[c], sem)
           for c in range(C)]
    for cp in cps:
        cp.wait()

    for c in range(C):
        for i in range(BPW // L):
            sl = pl.ds(i * L, L)
            gidx[c, sl] = gidx[c, sl] + c * V

    cps = [pltpu.async_copy(emb2d_hbm.at[gidx.at[c]], rows.at[c], sem)
           for c in range(C)]
    for cp in cps:
        cp.wait()

    for c in range(C):
        pltpu.sync_copy(rows.at[c], out_hbm.at[c, pl.ds(base, BPW)])


BB = 512


def _tc_project(g_ref, ue_ref, w_ref, b_ref, o_ref):
    acc = jnp.dot(ue_ref[...], w_ref[0:H, :], preferred_element_type=jnp.float32)
    for c in range(C):
        acc += jnp.dot(g_ref[c], w_ref[H + c * H:H + (c + 1) * H, :],
                       preferred_element_type=jnp.float32)
    o_ref[...] = acc + b_ref[...]


def kernel(uids, user_embedding, attr_table, embed_tables, W, b):
    attr_flat = attr_table.reshape(-1)
    emb2d = embed_tables.reshape(C * V, H)
    gathered = _sc_gather(uids, attr_flat, emb2d)

    out = pl.pallas_call(
        _tc_project,
        grid=(B // BB,),
        in_specs=[
            pl.BlockSpec((C, BB, H), lambda i: (0, i, 0)),
            pl.BlockSpec((BB, H), lambda i: (i, 0)),
            pl.BlockSpec((C * H + H, H), lambda i: (0, 0)),
            pl.BlockSpec((1, H), lambda i: (0, 0)),
        ],
        out_specs=pl.BlockSpec((BB, H), lambda i: (i, 0)),
        out_shape=jax.ShapeDtypeStruct((B, H), jnp.float32),
    )(gathered, user_embedding, W, b.reshape(1, H))
    return out


# speedup vs baseline: 4.1946x; 1.9776x over previous
import functools

import jax
import jax.numpy as jnp
from jax import lax
from jax.experimental import pallas as pl
from jax.experimental.pallas import tpu as pltpu
from jax.experimental.pallas import tpu_sc as plsc

B = 4096
C = 26
V = 100000
H = 32
NU = 100000

NC = 2
NS = 16
NW = NC * NS
BPW = B // NW

_mesh = plsc.VectorSubcoreMesh(core_axis_name="c", subcore_axis_name="s")


@functools.partial(
    pl.kernel,
    mesh=_mesh,
    out_type=jax.ShapeDtypeStruct((C, H, B), jnp.float32),
    scratch_types=[
        pltpu.VMEM((BPW,), jnp.int32),
        pltpu.VMEM((C, BPW), jnp.int32),
        pltpu.VMEM((2, H, BPW), jnp.float32),
        pltpu.SemaphoreType.DMA,
        pltpu.SemaphoreType.DMA,
        pltpu.SemaphoreType.DMA,
        pltpu.SemaphoreType.DMA,
    ],
)
def _sc_gather(uids_hbm, attr_t_hbm, emb_t_hbm, out_hbm,
               uids_v, attrs_v, colbuf, sem1, semg0, semg1, semw):
    wid = lax.axis_index("s") * NC + lax.axis_index("c")
    base = wid * BPW
    pltpu.sync_copy(uids_hbm.at[pl.ds(base, BPW)], uids_v)

    cps = [pltpu.async_copy(attr_t_hbm.at[pl.ds(c * NU, NU)].at[uids_v],
                            attrs_v.at[c], sem1)
           for c in range(C)]
    for cp in cps:
        cp.wait()

    semg = (semg0, semg1)

    def fire(c):
        buf = colbuf.at[c % 2]

        def body(h, carry):
            src = emb_t_hbm.at[pl.ds((c * H + h) * NU, NU)]
            pltpu.async_copy(src.at[attrs_v.at[c]], buf.at[h], semg[c % 2])
            return carry

        lax.fori_loop(0, H, body, 0)

    def complete(c):
        buf = colbuf.at[c % 2]
        dummy = out_hbm.at[0, :, pl.ds(0, BPW)]
        pltpu.make_async_copy(dummy, buf, semg[c % 2]).wait()
        pltpu.async_copy(buf, out_hbm.at[c, :, pl.ds(base, BPW)], semw)

    def drain_writeout(c):
        dummy = out_hbm.at[0, :, pl.ds(0, BPW)]
        pltpu.make_async_copy(dummy, colbuf.at[c % 2], semw).wait()

    fire(0)
    for c in range(1, C):
        if c >= 2:
            drain_writeout(c - 2)
        fire(c)
        complete(c - 1)
    complete(C - 1)
    drain_writeout(C - 2)
    drain_writeout(C - 1)


BB = 512


def _tc_project(g_ref, ue_ref, w_ref, b_ref, o_ref):
    acc = jnp.dot(ue_ref[...], w_ref[0:H, :], preferred_element_type=jnp.float32)
    for c in range(C):
        acc += lax.dot_general(
            g_ref[c], w_ref[H + c * H:H + (c + 1) * H, :],
            dimension_numbers=(((0,), (0,)), ((), ())),
            preferred_element_type=jnp.float32)
    o_ref[...] = acc + b_ref[...]


def kernel(uids, user_embedding, attr_table, embed_tables, W, b):
    attr_t = attr_table.T.reshape(-1)
    emb_t = embed_tables.transpose(0, 2, 1).reshape(-1)
    gathered = _sc_gather(uids, attr_t, emb_t)

    out = pl.pallas_call(
        _tc_project,
        grid=(B // BB,),
        in_specs=[
            pl.BlockSpec((C, H, BB), lambda i: (0, 0, i)),
            pl.BlockSpec((BB, H), lambda i: (i, 0)),
            pl.BlockSpec((C * H + H, H), lambda i: (0, 0)),
            pl.BlockSpec((1, H), lambda i: (0, 0)),
        ],
        out_specs=pl.BlockSpec((BB, H), lambda i: (i, 0)),
        out_shape=jax.ShapeDtypeStruct((B, H), jnp.float32),
    )(gathered, user_embedding, W, b.reshape(1, H))
    return out
